# baseline (speedup 1.0000x reference)
import functools

import jax
import jax.numpy as jnp
from jax import lax
from jax.experimental import pallas as pl
from jax.experimental.pallas import tpu as pltpu
from jax.experimental.pallas import tpu_sc as plsc

N = 10000
E = 320000
D_IN = 128
D_H = 16
D_OUT = 128
D_EDGE = 16
KH = 32
BN_EPS = 1e-5

NC, NS = 2, 16
NW = NC * NS
IDXB = 128
GROUP = 16
ROWS_W = 80
E_PAD = NW * ROWS_W * IDXB
PAD = E_PAD - E
N_ACC = 10240
N_P = 10240
BN_NODE = 1024
BE = 2048


def _node_prep_body(h_ref, sin_ref, bein_ref, w1_ref, b1_ref, p_ref):
    x = jnp.maximum(h_ref[...] * sin_ref[...] + bein_ref[...], 0.0)
    p = jnp.dot(x, w1_ref[...], preferred_element_type=jnp.float32)
    p_ref[...] = jnp.maximum(p + b1_ref[...], 0.0)


def _edge_body(m_ref, ea_ref, wk1_ref, bk1_ref, wk2_ref, bk2_ref, rm_ref,
               g_ref, msg_ref):
    m_p = m_ref[...]
    ea3 = ea_ref[...].reshape(BE // 8, 8, 16)
    ea_p = jnp.concatenate(
        [ea3[:, s, :] for s in range(8)], axis=1)
    kh_p = jnp.maximum(
        jnp.dot(ea_p, wk1_ref[...], preferred_element_type=jnp.float32)
        + bk1_ref[...], 0.0).astype(jnp.bfloat16)
    big = jnp.dot(kh_p, wk2_ref[...],
                  preferred_element_type=jnp.float32) + bk2_ref[...]
    mexp = jnp.dot(m_p.astype(jnp.bfloat16), rm_ref[...],
                   preferred_element_type=jnp.float32).astype(jnp.bfloat16)
    z = big[:, :2048].astype(jnp.bfloat16) * mexp
    msg_ref[...] = jnp.dot(z, g_ref[...],
                           preferred_element_type=jnp.float32) \
        + big[:, 2048:]


def _final_body(u0_ref, u1_ref, supd_ref, beupd_ref, w2_ref, b2_ref, o_ref):
    u = (u0_ref[...] + u1_ref[...]).reshape(BN_NODE, D_H)
    u = jnp.maximum(u * supd_ref[...] + beupd_ref[...], 0.0)
    o_ref[...] = jnp.dot(u, w2_ref[...],
                         preferred_element_type=jnp.float32) + b2_ref[...]


_MESH = plsc.VectorSubcoreMesh(core_axis_name="c", subcore_axis_name="s",
                               num_cores=NC, num_subcores=NS)


@functools.partial(
    pl.kernel, mesh=_MESH,
    compiler_params=pltpu.CompilerParams(use_tc_tiling_on_sc=False),
    out_type=jax.ShapeDtypeStruct((E_PAD, D_H), jnp.float32),
    scratch_types=[
        pltpu.VMEM((GROUP, IDXB), jnp.int32),
        pltpu.VMEM((GROUP * IDXB, D_H), jnp.float32),
        pltpu.VMEM_SHARED((N_P, D_H), jnp.float32),
        pltpu.SemaphoreType.DMA,
    ],
)
def _sc_gather(p_hbm, idx_hbm, out_hbm, idx_v, rows_v, p_sh, sem):
    cid = lax.axis_index("c")
    sid = lax.axis_index("s")
    wid = sid * NC + cid
    out_r = out_hbm

    @pl.when(sid == 0)
    def _stage_table():
        pltpu.sync_copy(p_hbm, p_sh)

    plsc.subcore_barrier()

    def body(g, carry):
        row0 = wid * ROWS_W + g * GROUP
        pltpu.sync_copy(idx_hbm.at[pl.ds(row0, GROUP)], idx_v)
        descs = [
            pltpu.async_copy(p_sh.at[idx_v.at[j]],
                             rows_v.at[pl.ds(j * IDXB, IDXB)], sem)
            for j in range(GROUP)
        ]
        for d in descs:
            d.wait()
        pltpu.sync_copy(rows_v, out_r.at[pl.ds(row0 * IDXB, GROUP * IDXB)])
        return carry

    lax.fori_loop(0, ROWS_W // GROUP, body, 0)


@functools.partial(
    pl.kernel, mesh=_MESH,
    compiler_params=pltpu.CompilerParams(use_tc_tiling_on_sc=False),
    out_type=jax.ShapeDtypeStruct((NC, N_ACC, D_H), jnp.float32),
    scratch_types=[
        pltpu.VMEM((GROUP, IDXB), jnp.int32),
        pltpu.VMEM((GROUP * IDXB, D_H), jnp.float32),
        pltpu.VMEM_SHARED((N_ACC, D_H), jnp.float32),
        pltpu.SemaphoreType.DMA,
    ],
)
def _sc_scatter(msg_hbm, dst_hbm, zeros_hbm, out_hbm, idx_v, rows_v, acc_sh,
                sem):
    cid = lax.axis_index("c")
    sid = lax.axis_index("s")
    wid = sid * NC + cid
    msg_r = msg_hbm
    out_r = out_hbm

    @pl.when(sid == 0)
    def _init():
        pltpu.sync_copy(zeros_hbm, acc_sh)

    plsc.subcore_barrier()

    def body(g, carry):
        row0 = wid * ROWS_W + g * GROUP
        pltpu.sync_copy(dst_hbm.at[pl.ds(row0, GROUP)], idx_v)
        pltpu.sync_copy(msg_r.at[pl.ds(row0 * IDXB, GROUP * IDXB)], rows_v)
        for j in range(GROUP):
            pltpu.sync_copy(rows_v.at[pl.ds(j * IDXB, IDXB)],
                            acc_sh.at[idx_v.at[j]], add=True)
        return carry

    lax.fori_loop(0, ROWS_W // GROUP, body, 0)
    plsc.subcore_barrier()

    rows_per_tile = N_ACC // NS
    pltpu.sync_copy(acc_sh.at[pl.ds(sid * rows_per_tile, rows_per_tile)],
                    out_r.at[cid, pl.ds(sid * rows_per_tile, rows_per_tile)])


def kernel(H, edges, edge_attr, W1, b1, Wk1, bk1, Wk2, bk2, W2, b2,
           g_in, be_in, g_msg, be_msg, g_upd, be_upd, g_out, be_out):
    f32 = jnp.float32
    inv = 1.0 / jnp.sqrt(jnp.asarray(1.0 + BN_EPS, f32))
    s_in = (g_in * inv)[None, :]
    s_upd = (g_upd * inv)[None, :]
    s_msg = g_msg * inv
    s_out = g_out * inv
    W1sT = (W1 * s_msg[:, None]).T
    b1s = (b1 * s_msg + be_msg)[None, :]
    Wk1T = Wk1.T
    bk1b = bk1[None, :]
    Wk2T = Wk2.T
    Wk2Tp = jnp.concatenate([Wk2T[:, 16:], Wk2T[:, :16]], axis=1)
    bk2p = jnp.concatenate([bk2[16:], bk2[:16]])[None, :]
    Rm = jnp.tile(jnp.eye(D_H, dtype=f32), (1, 16))
    G = jnp.kron(jnp.eye(D_H, dtype=f32), jnp.ones((16, 1), f32))
    W2sT = (W2 * s_out[:, None]).T
    b2s = (b2 * s_out + be_out)[None, :]

    idx_in = jnp.concatenate(
        [edges[1], jnp.zeros((PAD,), jnp.int32)]).reshape(NW * ROWS_W, IDXB)
    dst = jnp.concatenate(
        [edges[0], N + (jnp.arange(PAD, dtype=jnp.int32) % (N_ACC - N))]
    ).reshape(NW * ROWS_W, IDXB)
    ea_pad = jnp.concatenate(
        [edge_attr, jnp.zeros((PAD, edge_attr.shape[1]), f32)])
    zeros_acc = jnp.zeros((N_ACC, D_H), f32)

    rep = lambda shape: pl.BlockSpec(shape, lambda i: (0, 0))
    H_pad = jnp.concatenate([H, jnp.zeros((N_P - N, D_IN), f32)])
    H_pk = H_pad.reshape(N_P // 8, 8 * D_IN)
    W1bd = jnp.kron(jnp.eye(8, dtype=f32), W1sT)
    s_in8 = jnp.tile(s_in, (1, 8))
    be_in8 = jnp.tile(be_in[None, :], (1, 8))
    b1s8 = jnp.tile(b1s, (1, 8))
    P_packed = pl.pallas_call(
        _node_prep_body,
        grid=(N_P // BN_NODE,),
        in_specs=[
            pl.BlockSpec((BN_NODE // 8, 8 * D_IN), lambda i: (i, 0)),
            rep((1, 8 * D_IN)), rep((1, 8 * D_IN)),
            rep((8 * D_IN, 8 * D_H)), rep((1, 8 * D_H)),
        ],
        out_specs=pl.BlockSpec((BN_NODE // 8, 8 * D_H), lambda i: (i, 0)),
        out_shape=jax.ShapeDtypeStruct((N_P // 8, 8 * D_H), f32),
    )(H_pk, s_in8, be_in8, W1bd, b1s8)
    P = P_packed.reshape(N_P, D_H)

    m_raw = _sc_gather(P, idx_in)
    m_packed = m_raw.reshape(E_PAD // 8, 128)

    bf16 = jnp.bfloat16
    Wk1bd = jnp.kron(jnp.eye(8, dtype=f32), Wk1T)
    bk1b8 = jnp.tile(bk1b, (1, 8))
    Wk2bd = jnp.concatenate(
        [jnp.kron(jnp.eye(8, dtype=f32), Wk2Tp[:, :256]),
         jnp.kron(jnp.eye(8, dtype=f32), Wk2Tp[:, 256:])],
        axis=1).astype(bf16)
    bk2bd = jnp.concatenate(
        [jnp.tile(bk2p[:, :256], (1, 8)), jnp.tile(bk2p[:, 256:], (1, 8))],
        axis=1)
    Rmbd = jnp.kron(jnp.eye(8, dtype=f32), Rm).astype(bf16)
    Gbd = jnp.kron(jnp.eye(8, dtype=f32), G).astype(bf16)
    msg_packed = pl.pallas_call(
        _edge_body,
        grid=(E_PAD // BE,),
        in_specs=[
            pl.BlockSpec((BE // 8, 128), lambda i: (i, 0)),
            pl.BlockSpec((BE, 16), lambda i: (i, 0)),
            rep((128, 8 * KH)), rep((1, 8 * KH)), rep((8 * KH, 2176)),
            rep((1, 2176)), rep((128, 2048)), rep((2048, 128)),
        ],
        out_specs=pl.BlockSpec((BE // 8, 128), lambda i: (i, 0)),
        out_shape=jax.ShapeDtypeStruct((E_PAD // 8, 128), f32),
    )(m_packed, ea_pad, Wk1bd, bk1b8, Wk2bd, bk2bd, Rmbd, Gbd)

    msg = msg_packed.reshape(E_PAD, D_H)
    parts = _sc_scatter(msg, dst, zeros_acc)
    parts_packed = parts.reshape(NC, N_ACC // 8, 8, D_H)

    rep3 = lambda shape: pl.BlockSpec(shape, lambda i: (i, 0, 0))
    out = pl.pallas_call(
        _final_body,
        grid=(N_ACC // BN_NODE,),
        in_specs=[
            rep3((BN_NODE // 8, 8, D_H)),
            rep3((BN_NODE // 8, 8, D_H)),
            rep((1, D_H)), rep((1, D_H)), rep((D_H, D_OUT)), rep((1, D_OUT)),
        ],
        out_specs=pl.BlockSpec((BN_NODE, D_OUT), lambda i: (i, 0)),
        out_shape=jax.ShapeDtypeStruct((N_ACC, D_OUT), f32),
    )(parts_packed[0], parts_packed[1], s_upd, be_upd[None, :],
      W2sT, b2s)
    return out[:N]


# speedup vs baseline: 1.1407x; 1.1407x over previous
import functools

import jax
import jax.numpy as jnp
from jax import lax
from jax.experimental import pallas as pl
from jax.experimental.pallas import tpu as pltpu
from jax.experimental.pallas import tpu_sc as plsc

N = 10000
E = 320000
D_IN = 128
D_H = 16
D_OUT = 128
D_EDGE = 16
KH = 32
BN_EPS = 1e-5

NC, NS = 2, 16
NW = NC * NS
IDXB = 128
GROUP = 16
ROWS_W = 80
E_PAD = NW * ROWS_W * IDXB
PAD = E_PAD - E
N_ACC = 10240
N_P = 10240
BN_NODE = 1024
BE = 2048


def _node_prep_body(h_ref, sin_ref, bein_ref, w1_ref, b1_ref, p_ref):
    x = jnp.maximum(h_ref[...] * sin_ref[...] + bein_ref[...], 0.0)
    p = jnp.dot(x, w1_ref[...], preferred_element_type=jnp.float32)
    p_ref[...] = jnp.maximum(p + b1_ref[...], 0.0)


def _edge_body(m_ref, ea_ref, wk1_ref, bk1_ref, wk2_ref, bk2_ref, rm_ref,
               g_ref, msg_ref):
    m_p = m_ref[...]
    ea_p = ea_ref[...]
    kh_p = jnp.maximum(
        jnp.dot(ea_p, wk1_ref[...], preferred_element_type=jnp.float32)
        + bk1_ref[...], 0.0).astype(jnp.bfloat16)
    big = jnp.dot(kh_p, wk2_ref[...],
                  preferred_element_type=jnp.float32) + bk2_ref[...]
    mexp = jnp.dot(m_p.astype(jnp.bfloat16), rm_ref[...],
                   preferred_element_type=jnp.float32).astype(jnp.bfloat16)
    z = big[:, :2048].astype(jnp.bfloat16) * mexp
    msg_ref[...] = jnp.dot(z, g_ref[...],
                           preferred_element_type=jnp.float32) \
        + big[:, 2048:]


def _final_body(u0_ref, u1_ref, supd_ref, beupd_ref, w2_ref, b2_ref, o_ref):
    u = (u0_ref[...] + u1_ref[...]).reshape(BN_NODE, D_H)
    u = jnp.maximum(u * supd_ref[...] + beupd_ref[...], 0.0)
    o_ref[...] = jnp.dot(u, w2_ref[...],
                         preferred_element_type=jnp.float32) + b2_ref[...]


_MESH = plsc.VectorSubcoreMesh(core_axis_name="c", subcore_axis_name="s",
                               num_cores=NC, num_subcores=NS)


@functools.partial(
    pl.kernel, mesh=_MESH,
    compiler_params=pltpu.CompilerParams(use_tc_tiling_on_sc=False),
    out_type=jax.ShapeDtypeStruct((E_PAD, D_H), jnp.float32),
    scratch_types=[
        pltpu.VMEM((GROUP, IDXB), jnp.int32),
        pltpu.VMEM((GROUP * IDXB, D_H), jnp.float32),
        pltpu.VMEM_SHARED((N_P, D_H), jnp.float32),
        pltpu.SemaphoreType.DMA,
    ],
)
def _sc_gather(p_hbm, idx_hbm, out_hbm, idx_v, rows_v, p_sh, sem):
    cid = lax.axis_index("c")
    sid = lax.axis_index("s")
    wid = sid * NC + cid
    out_r = out_hbm

    @pl.when(sid == 0)
    def _stage_table():
        pltpu.sync_copy(p_hbm, p_sh)

    plsc.subcore_barrier()

    def body(g, carry):
        row0 = wid * ROWS_W + g * GROUP
        pltpu.sync_copy(idx_hbm.at[pl.ds(row0, GROUP)], idx_v)
        descs = [
            pltpu.async_copy(p_sh.at[idx_v.at[j]],
                             rows_v.at[pl.ds(j * IDXB, IDXB)], sem)
            for j in range(GROUP)
        ]
        for d in descs:
            d.wait()
        pltpu.sync_copy(rows_v, out_r.at[pl.ds(row0 * IDXB, GROUP * IDXB)])
        return carry

    lax.fori_loop(0, ROWS_W // GROUP, body, 0)


@functools.partial(
    pl.kernel, mesh=_MESH,
    compiler_params=pltpu.CompilerParams(use_tc_tiling_on_sc=False),
    out_type=jax.ShapeDtypeStruct((NC, N_ACC, D_H), jnp.float32),
    scratch_types=[
        pltpu.VMEM((GROUP, IDXB), jnp.int32),
        pltpu.VMEM((GROUP * IDXB, D_H), jnp.float32),
        pltpu.VMEM_SHARED((N_ACC, D_H), jnp.float32),
        pltpu.SemaphoreType.DMA,
    ],
)
def _sc_scatter(msg_hbm, dst_hbm, zeros_hbm, out_hbm, idx_v, rows_v, acc_sh,
                sem):
    cid = lax.axis_index("c")
    sid = lax.axis_index("s")
    wid = sid * NC + cid
    msg_r = msg_hbm
    out_r = out_hbm

    @pl.when(sid == 0)
    def _init():
        pltpu.sync_copy(zeros_hbm, acc_sh)

    plsc.subcore_barrier()

    def body(g, carry):
        row0 = wid * ROWS_W + g * GROUP
        pltpu.sync_copy(dst_hbm.at[pl.ds(row0, GROUP)], idx_v)
        pltpu.sync_copy(msg_r.at[pl.ds(row0 * IDXB, GROUP * IDXB)], rows_v)
        for j in range(GROUP):
            pltpu.sync_copy(rows_v.at[pl.ds(j * IDXB, IDXB)],
                            acc_sh.at[idx_v.at[j]], add=True)
        return carry

    lax.fori_loop(0, ROWS_W // GROUP, body, 0)
    plsc.subcore_barrier()

    rows_per_tile = N_ACC // NS
    pltpu.sync_copy(acc_sh.at[pl.ds(sid * rows_per_tile, rows_per_tile)],
                    out_r.at[cid, pl.ds(sid * rows_per_tile, rows_per_tile)])


def kernel(H, edges, edge_attr, W1, b1, Wk1, bk1, Wk2, bk2, W2, b2,
           g_in, be_in, g_msg, be_msg, g_upd, be_upd, g_out, be_out):
    f32 = jnp.float32
    inv = 1.0 / jnp.sqrt(jnp.asarray(1.0 + BN_EPS, f32))
    s_in = (g_in * inv)[None, :]
    s_upd = (g_upd * inv)[None, :]
    s_msg = g_msg * inv
    s_out = g_out * inv
    W1sT = (W1 * s_msg[:, None]).T
    b1s = (b1 * s_msg + be_msg)[None, :]
    Wk1T = Wk1.T
    bk1b = bk1[None, :]
    Wk2T = Wk2.T
    Wk2Tp = jnp.concatenate([Wk2T[:, 16:], Wk2T[:, :16]], axis=1)
    bk2p = jnp.concatenate([bk2[16:], bk2[:16]])[None, :]
    Rm = jnp.tile(jnp.eye(D_H, dtype=f32), (1, 16))
    G = jnp.kron(jnp.eye(D_H, dtype=f32), jnp.ones((16, 1), f32))
    W2sT = (W2 * s_out[:, None]).T
    b2s = (b2 * s_out + be_out)[None, :]

    idx_in = jnp.concatenate(
        [edges[1], jnp.zeros((PAD,), jnp.int32)]).reshape(NW * ROWS_W, IDXB)
    dst = jnp.concatenate(
        [edges[0], N + (jnp.arange(PAD, dtype=jnp.int32) % (N_ACC - N))]
    ).reshape(NW * ROWS_W, IDXB)
    ea_packed = jnp.concatenate(
        [edge_attr.reshape(E // 8, 128), jnp.zeros((PAD // 8, 128), f32)])
    zeros_acc = jnp.zeros((N_ACC, D_H), f32)

    rep = lambda shape: pl.BlockSpec(shape, lambda i: (0, 0))
    H_pad = jnp.concatenate([H, jnp.zeros((N_P - N, D_IN), f32)])
    H_pk = H_pad.reshape(N_P // 8, 8 * D_IN)
    W1bd = jnp.kron(jnp.eye(8, dtype=f32), W1sT)
    s_in8 = jnp.tile(s_in, (1, 8))
    be_in8 = jnp.tile(be_in[None, :], (1, 8))
    b1s8 = jnp.tile(b1s, (1, 8))
    P_packed = pl.pallas_call(
        _node_prep_body,
        grid=(N_P // BN_NODE,),
        in_specs=[
            pl.BlockSpec((BN_NODE // 8, 8 * D_IN), lambda i: (i, 0)),
            rep((1, 8 * D_IN)), rep((1, 8 * D_IN)),
            rep((8 * D_IN, 8 * D_H)), rep((1, 8 * D_H)),
        ],
        out_specs=pl.BlockSpec((BN_NODE // 8, 8 * D_H), lambda i: (i, 0)),
        out_shape=jax.ShapeDtypeStruct((N_P // 8, 8 * D_H), f32),
    )(H_pk, s_in8, be_in8, W1bd, b1s8)
    P = P_packed.reshape(N_P, D_H)

    m_raw = _sc_gather(P, idx_in)
    m_packed = m_raw.reshape(E_PAD // 8, 128)

    bf16 = jnp.bfloat16
    Wk1bd = jnp.kron(jnp.eye(8, dtype=f32), Wk1T)
    bk1b8 = jnp.tile(bk1b, (1, 8))
    Wk2bd = jnp.concatenate(
        [jnp.kron(jnp.eye(8, dtype=f32), Wk2Tp[:, :256]),
         jnp.kron(jnp.eye(8, dtype=f32), Wk2Tp[:, 256:])],
        axis=1).astype(bf16)
    bk2bd = jnp.concatenate(
        [jnp.tile(bk2p[:, :256], (1, 8)), jnp.tile(bk2p[:, 256:], (1, 8))],
        axis=1)
    Rmbd = jnp.kron(jnp.eye(8, dtype=f32), Rm).astype(bf16)
    Gbd = jnp.kron(jnp.eye(8, dtype=f32), G).astype(bf16)
    msg_packed = pl.pallas_call(
        _edge_body,
        grid=(E_PAD // BE,),
        in_specs=[
            pl.BlockSpec((BE // 8, 128), lambda i: (i, 0)),
            pl.BlockSpec((BE // 8, 128), lambda i: (i, 0)),
            rep((128, 8 * KH)), rep((1, 8 * KH)), rep((8 * KH, 2176)),
            rep((1, 2176)), rep((128, 2048)), rep((2048, 128)),
        ],
        out_specs=pl.BlockSpec((BE // 8, 128), lambda i: (i, 0)),
        out_shape=jax.ShapeDtypeStruct((E_PAD // 8, 128), f32),
    )(m_packed, ea_packed, Wk1bd, bk1b8, Wk2bd, bk2bd, Rmbd, Gbd)

    msg = msg_packed.reshape(E_PAD, D_H)
    parts = _sc_scatter(msg, dst, zeros_acc)
    parts_packed = parts.reshape(NC, N_ACC // 8, 8, D_H)

    rep3 = lambda shape: pl.BlockSpec(shape, lambda i: (i, 0, 0))
    out = pl.pallas_call(
        _final_body,
        grid=(N_ACC // BN_NODE,),
        in_specs=[
            rep3((BN_NODE // 8, 8, D_H)),
            rep3((BN_NODE // 8, 8, D_H)),
            rep((1, D_H)), rep((1, D_H)), rep((D_H, D_OUT)), rep((1, D_OUT)),
        ],
        out_specs=pl.BlockSpec((BN_NODE, D_OUT), lambda i: (i, 0)),
        out_shape=jax.ShapeDtypeStruct((N_ACC, D_OUT), f32),
    )(parts_packed[0], parts_packed[1], s_upd, be_upd[None, :],
      W2sT, b2s)
    return out[:N]
